# baseline (speedup 1.0000x reference)
import functools

import jax
import jax.numpy as jnp
from jax import lax
from jax.experimental import pallas as pl
from jax.experimental.pallas import tpu as pltpu
from jax.experimental.pallas import tpu_sc as plsc

B = 16384
D = 64
N = 1000000
NBLK = (N + 1023) // 1024
NP = NBLK * 512

NC = 2
NS = 16
NW = NC * NS
BPW = B // NW
CH = BPW // 2


def _tr_body(tt_ref, out_ref):
    x = tt_ref[...]
    y = jnp.transpose(x, (1, 0))
    out_ref[...] = jnp.concatenate([y[:512], y[512:]], axis=1)


def _pack_rows(tt):
    return pl.pallas_call(
        _tr_body,
        grid=(NBLK,),
        in_specs=[pl.BlockSpec((D, 1024), lambda r: (0, r))],
        out_specs=pl.BlockSpec((512, 128), lambda r: (r, 0)),
        out_shape=jax.ShapeDtypeStruct((NP, 128), jnp.float32),
    )(tt)


def _sc_gather_body(idx_hbm, tab, out, idx_v, buf0, buf1, sg0, sg1, so0, so1):
    wid = lax.axis_index("s") * NC + lax.axis_index("c")
    base = wid * BPW
    pltpu.sync_copy(idx_hbm.at[pl.ds(base, BPW)], idx_v)
    for c in range(BPW // 16):
        sl = pl.ds(c * 16, 16)
        v = idx_v[sl]
        idx_v[sl] = ((v >> 10) << 9) | (v & 511)
    g0 = pltpu.async_copy(tab.at[idx_v.at[pl.ds(0, CH)]], buf0, sg0)
    g1 = pltpu.async_copy(tab.at[idx_v.at[pl.ds(CH, CH)]], buf1, sg1)
    g0.wait()
    o0 = pltpu.async_copy(buf0, out.at[pl.ds(base, CH)], so0)
    g1.wait()
    o1 = pltpu.async_copy(buf1, out.at[pl.ds(base + CH, CH)], so1)
    o0.wait()
    o1.wait()


@functools.cache
def _sc_gather():
    mesh = plsc.VectorSubcoreMesh(
        core_axis_name="c", subcore_axis_name="s", num_cores=NC, num_subcores=NS
    )
    return pl.kernel(
        _sc_gather_body,
        out_type=jax.ShapeDtypeStruct((B, 128), jnp.float32),
        mesh=mesh,
        scratch_types=[
            pltpu.VMEM((BPW,), jnp.int32),
            pltpu.VMEM((CH, 128), jnp.float32),
            pltpu.VMEM((CH, 128), jnp.float32),
            pltpu.SemaphoreType.DMA,
            pltpu.SemaphoreType.DMA,
            pltpu.SemaphoreType.DMA,
            pltpu.SemaphoreType.DMA,
        ],
    )


def _half(x, par):
    return x[:, :D] * (1.0 - par) + x[:, D:] * par


def _tc_body(up_ref, ip_ref, gu_ref, gi_ref, mu_ref, mi_ref,
             w1_ref, b1_ref, w2_ref, b2_ref, w3_ref, b3_ref,
             wp_ref, bp_ref, out_ref):
    upar = up_ref[...].astype(jnp.float32)
    ipar = ip_ref[...].astype(jnp.float32)
    mu = _half(mu_ref[...], upar)
    mi = _half(mi_ref[...], ipar)
    gu = _half(gu_ref[...], upar)
    gi = _half(gi_ref[...], ipar)
    w1 = w1_ref[...]
    h = jnp.dot(mu, w1[:D], preferred_element_type=jnp.float32)
    h = h + jnp.dot(mi, w1[D:], preferred_element_type=jnp.float32)
    h = jnp.maximum(h + b1_ref[...], 0.0)
    h = jnp.maximum(
        jnp.dot(h, w2_ref[...], preferred_element_type=jnp.float32) + b2_ref[...], 0.0)
    h = jnp.maximum(
        jnp.dot(h, w3_ref[...], preferred_element_type=jnp.float32) + b3_ref[...], 0.0)
    g = gu * gi
    wp = wp_ref[...]
    pred = jnp.dot(g, wp[:D], preferred_element_type=jnp.float32)
    pred = pred + jnp.dot(h, wp[D:], preferred_element_type=jnp.float32)
    out_ref[...] = pred + bp_ref[...]


def _tc_dense(upar, ipar, gu, gi, mu, mi, W1, b1, W2, b2, W3, b3, Wp, bp):
    R = 2048
    grid = (B // R,)
    row_spec = pl.BlockSpec((R, 128), lambda r: (r, 0))
    par_spec = pl.BlockSpec((R, 1), lambda r: (r, 0))

    def full(shape):
        return pl.BlockSpec(shape, lambda r: (0,) * len(shape))

    return pl.pallas_call(
        _tc_body,
        grid=grid,
        in_specs=[
            par_spec, par_spec,
            row_spec, row_spec, row_spec, row_spec,
            full(W1.shape), full((1, b1.shape[0])),
            full(W2.shape), full((1, b2.shape[0])),
            full(W3.shape), full((1, b3.shape[0])),
            full(Wp.shape), full((1, 1)),
        ],
        out_specs=pl.BlockSpec((R, 1), lambda r: (r, 0)),
        out_shape=jax.ShapeDtypeStruct((B, 1), jnp.float32),
    )(upar, ipar, gu, gi, mu, mi, W1, b1.reshape(1, -1), W2, b2.reshape(1, -1),
      W3, b3.reshape(1, -1), Wp, bp.reshape(1, 1))


def kernel(u, i, gmf_user_table, gmf_item_table, mlp_user_table, mlp_item_table,
           W1, b1, W2, b2, W3, b3, Wp, bp):
    u = u.astype(jnp.int32)
    i = i.astype(jnp.int32)
    gather = _sc_gather()
    packed = [
        _pack_rows(jnp.swapaxes(t, 0, 1))
        for t in (gmf_user_table, gmf_item_table, mlp_user_table, mlp_item_table)
    ]
    gu = gather(u, packed[0])
    gi = gather(i, packed[1])
    mu = gather(u, packed[2])
    mi = gather(i, packed[3])
    upar = ((u >> 9) & 1).reshape(B, 1)
    ipar = ((i >> 9) & 1).reshape(B, 1)
    out = _tc_dense(upar, ipar, gu, gi, mu, mi,
                    W1, b1, W2, b2, W3, b3, Wp, bp)
    return out[:, 0]
